# Initial kernel scaffold; baseline (speedup 1.0000x reference)
import jax
import jax.numpy as jnp
from jax.experimental import pallas as pl


def kernel(x, edge_index, edge_attr, Wi, bi, W1, b1, g1, be1, W2, b2, g2, be2, W3, b3, g3, be3, Wo, bo):
    raise NotImplementedError("write your pallas kernel here")


# speedup vs baseline: 6.2498x; 6.2498x over previous
import dataclasses
import functools

import jax
import jax.numpy as jnp
from jax import lax
from jax.experimental import pallas as pl
from jax.experimental.pallas import tpu as pltpu
from jax.experimental.pallas import tpu_sc as plsc

N = 10000
E = 320000
NC_SC = 2
NS_SC = 16
NW = NC_SC * NS_SC
CHUNK = 128
NCHUNK = 79
EPT = CHUNK * NCHUNK
E_PAD = EPT * NW
NPAD = 10240
RPT = NPAD // NW
RPS = NPAD // NS_SC


def _sc_mesh():
    return plsc.VectorSubcoreMesh(core_axis_name="c", subcore_axis_name="s")


def _sc_params():
    cp = pltpu.CompilerParams()
    if "needs_layout_passes" in pltpu.CompilerParams.__dataclass_fields__:
        cp = dataclasses.replace(cp, needs_layout_passes=False)
    return cp


def _sc_degree(w_rows, dstp):
    @functools.partial(
        pl.kernel,
        out_type=jax.ShapeDtypeStruct((NC_SC, NPAD, 16), jnp.float32),
        mesh=_sc_mesh(),
        scratch_types=[
            pltpu.VMEM((CHUNK, 16), jnp.float32),
            pltpu.VMEM((CHUNK,), jnp.int32),
            pltpu.VMEM_SHARED((NPAD, 16), jnp.float32),
        ],
    )
    def k(wrows_hbm, dst_hbm, out_hbm, wbuf, didx, acc):
        cid = lax.axis_index("c")
        sid = lax.axis_index("s")
        wid = cid * NS_SC + sid

        @pl.loop(0, CHUNK)
        def _(r):
            wbuf[r, :] = jnp.zeros((16,), jnp.float32)

        @pl.loop(0, RPS // CHUNK)
        def _(j):
            pltpu.sync_copy(wbuf, acc.at[pl.ds(sid * RPS + j * CHUNK, CHUNK)])

        plsc.subcore_barrier()
        base = wid * EPT

        @pl.loop(0, NCHUNK)
        def _(i):
            off = base + i * CHUNK
            pltpu.sync_copy(dst_hbm.at[pl.ds(off, CHUNK)], didx)
            pltpu.sync_copy(wrows_hbm.at[pl.ds(off, CHUNK)], wbuf)
            pltpu.sync_copy(wbuf, acc.at[didx], add=True)

        plsc.subcore_barrier()

        @pl.loop(0, RPS // CHUNK)
        def _(j):
            r0 = sid * RPS + j * CHUNK
            pltpu.sync_copy(acc.at[pl.ds(r0, CHUNK)],
                            out_hbm.at[cid, pl.ds(r0, CHUNK)])

    return k(w_rows, dstp)


def _sc_message(y, srcp, dstp, wp, F):
    FC = F // 16

    @functools.partial(
        pl.kernel,
        out_type=jax.ShapeDtypeStruct((NC_SC, NPAD, F), jnp.float32),
        mesh=_sc_mesh(),
        scratch_types=[
            pltpu.VMEM((CHUNK, F), jnp.float32),
            pltpu.VMEM((CHUNK,), jnp.int32),
            pltpu.VMEM((CHUNK,), jnp.int32),
            pltpu.VMEM((CHUNK,), jnp.float32),
            pltpu.VMEM_SHARED((NPAD, F), jnp.float32),
            pltpu.SemaphoreType.DMA,
        ],
        compiler_params=_sc_params(),
    )
    def k(y_hbm, src_hbm, dst_hbm, w_hbm, out_hbm, rows, sidx, didx, wv, acc,
          sem):
        cid = lax.axis_index("c")
        sid = lax.axis_index("s")
        wid = cid * NS_SC + sid

        @pl.loop(0, CHUNK)
        def _(r):
            for c in range(FC):
                rows[r, pl.ds(c * 16, 16)] = jnp.zeros((16,), jnp.float32)

        @pl.loop(0, RPS // CHUNK)
        def _(j):
            pltpu.sync_copy(rows, acc.at[pl.ds(sid * RPS + j * CHUNK, CHUNK)])

        plsc.subcore_barrier()
        base = wid * EPT

        @pl.loop(0, NCHUNK)
        def _(i):
            off = base + i * CHUNK
            pltpu.sync_copy(src_hbm.at[pl.ds(off, CHUNK)], sidx)
            pltpu.sync_copy(dst_hbm.at[pl.ds(off, CHUNK)], didx)
            pltpu.sync_copy(w_hbm.at[pl.ds(off, CHUNK)], wv)
            pltpu.async_copy(y_hbm.at[sidx], rows, sem).wait()

            @pl.loop(0, CHUNK)
            def _(b):
                bvec = jnp.full((16,), b, jnp.int32)
                ws = plsc.load_gather(wv, [bvec])
                for c in range(FC):
                    cur = rows[b, pl.ds(c * 16, 16)]
                    rows[b, pl.ds(c * 16, 16)] = cur * ws

            pltpu.sync_copy(rows, acc.at[didx], add=True)

        plsc.subcore_barrier()

        @pl.loop(0, RPS // CHUNK)
        def _(j):
            r0 = sid * RPS + j * CHUNK
            pltpu.sync_copy(acc.at[pl.ds(r0, CHUNK)],
                            out_hbm.at[cid, pl.ds(r0, CHUNK)])

    return k(y, srcp, dstp, wp)


def _tc_prologue(x, WiT, bi, W1T, degp):
    def f(x_ref, wit_ref, bi_ref, w1t_ref, deg_ref, y1_ref, dis_ref):
        h0 = jnp.maximum(
            jnp.dot(x_ref[...], wit_ref[...],
                    preferred_element_type=jnp.float32) + bi_ref[...], 0.0)
        deg = deg_ref[0, :N, 0] + deg_ref[1, :N, 0] + 1.0
        dis = lax.rsqrt(deg)
        xw = jnp.dot(h0, w1t_ref[...], preferred_element_type=jnp.float32)
        y1_ref[...] = xw * dis[:, None]
        dis_ref[...] = dis

    return pl.pallas_call(
        f,
        out_shape=(
            jax.ShapeDtypeStruct((N, 128), jnp.float32),
            jax.ShapeDtypeStruct((N,), jnp.float32),
        ),
    )(x, WiT, bi, W1T, degp)


def _tc_mid(accp, y, dis, b, g, be, WnT, Fn):
    def f(acc_ref, y_ref, dis_ref, b_ref, g_ref, be_ref, wnt_ref, yn_ref):
        dis = dis_ref[...]
        t = (acc_ref[0, :N, :] + acc_ref[1, :N, :] + y_ref[...])
        t = t * dis[:, None] + b_ref[...]
        m = jnp.mean(t, axis=0)
        v = jnp.mean((t - m[None, :]) ** 2, axis=0)
        h = (t - m[None, :]) * lax.rsqrt(v[None, :] + 1e-5)
        h = jnp.maximum(h * g_ref[...] + be_ref[...], 0.0)
        xw = jnp.dot(h, wnt_ref[...], preferred_element_type=jnp.float32)
        yn = xw * dis[:, None]
        if Fn < 128:
            yn = jnp.concatenate(
                [yn, jnp.zeros((N, 128 - Fn), jnp.float32)], axis=1)
        yn_ref[...] = yn

    return pl.pallas_call(
        f,
        out_shape=jax.ShapeDtypeStruct((N, 128), jnp.float32),
    )(accp, y, dis, b, g, be, WnT)


def _tc_epilogue(accp, y, dis, b3, g3, be3, WoT, bo):
    def f(acc_ref, y_ref, dis_ref, b_ref, g_ref, be_ref, wot_ref, bo_ref,
          out_ref):
        dis = dis_ref[...]
        t = (acc_ref[0, :N, :64] + acc_ref[1, :N, :64] + y_ref[:, :64])
        t = t * dis[:, None] + b_ref[...]
        m = jnp.mean(t, axis=0)
        v = jnp.mean((t - m[None, :]) ** 2, axis=0)
        h = (t - m[None, :]) * lax.rsqrt(v[None, :] + 1e-5)
        h = jnp.maximum(h * g_ref[...] + be_ref[...], 0.0)
        logits = jnp.dot(h, wot_ref[...],
                         preferred_element_type=jnp.float32) + bo_ref[...]
        mx = jnp.max(logits, axis=1, keepdims=True)
        s = logits - mx
        lse = jnp.log(jnp.sum(jnp.exp(s), axis=1, keepdims=True))
        out_ref[...] = s - lse

    return pl.pallas_call(
        f,
        out_shape=jax.ShapeDtypeStruct((N, 10), jnp.float32),
    )(accp, y, dis, b3, g3, be3, WoT, bo)


def kernel(x, edge_index, edge_attr, Wi, bi, W1, b1, g1, be1, W2, b2, g2, be2,
           W3, b3, g3, be3, Wo, bo):
    src = edge_index[0].astype(jnp.int32)
    dst = edge_index[1].astype(jnp.int32)
    w = edge_attr.astype(jnp.float32)

    pad = E_PAD - E
    srcp = jnp.concatenate([src, jnp.zeros((pad,), jnp.int32)])
    dstp = jnp.concatenate([dst, jnp.zeros((pad,), jnp.int32)])
    wp = jnp.concatenate([w, jnp.zeros((pad,), jnp.float32)])
    w_rows = jnp.zeros((E_PAD, 16), jnp.float32).at[:, 0].set(wp)

    degp = _sc_degree(w_rows, dstp)

    y1, dis = _tc_prologue(x, Wi.T, bi, W1.T, degp)

    acc1 = _sc_message(y1, srcp, dstp, wp, 128)
    y2 = _tc_mid(acc1, y1, dis, b1, g1, be1, W2.T, 128)

    acc2 = _sc_message(y2, srcp, dstp, wp, 128)
    y3 = _tc_mid(acc2, y2, dis, b2, g2, be2, W3.T, 64)

    acc3 = _sc_message(y3, srcp, dstp, wp, 128)
    out = _tc_epilogue(acc3, y3, dis, b3, g3, be3, Wo.T, bo)
    return out
